# baseline (speedup 1.0000x reference)
import functools

import jax
import jax.numpy as jnp
from jax import lax
from jax.experimental import pallas as pl
from jax.experimental.pallas import tpu as pltpu
from jax.experimental.pallas import tpu_sc as plsc

E = 64
D_IN = 768
D_OUT = 768
T = 8192
BLK = 256
NBLK = T // BLK + E
TPAD = NBLK * BLK

NW = 32
TOK_PER_W = T // NW
CHUNK = 64
NCHUNK = TOK_PER_W // CHUNK


def _cumsum_lanes(x, n):
    i = lax.broadcasted_iota(jnp.int32, x.shape, len(x.shape) - 1)
    k = 1
    while k < n:
        r = pltpu.roll(x, k, axis=len(x.shape) - 1)
        x = x + jnp.where(i >= k, r, 0)
        k *= 2
    return x


def _cumsum_sublanes(x, n):
    i = lax.broadcasted_iota(jnp.int32, x.shape, 0)
    k = 1
    while k < n:
        r = pltpu.roll(x, k, axis=0)
        x = x + jnp.where(i >= k, r, 0)
        k *= 2
    return x


def _prep_body(gate_ref, dst_ref, be_ref, tot_ref, xy_ref):
    gate = gate_ref[...]
    e_col = lax.broadcasted_iota(jnp.int32, (E, T), 0)
    onehot = (gate == e_col).astype(jnp.int32)
    cum = _cumsum_lanes(onehot, T)
    rank = jnp.sum(onehot * cum, axis=0, keepdims=True) - 1
    counts = cum[:, T - 1:T]
    blk_counts = (counts + BLK - 1) // BLK
    blk_end = _cumsum_sublanes(blk_counts, E)
    blk_start = blk_end - blk_counts
    pad_start = blk_start * BLK
    dst_ref[...] = jnp.sum(onehot * pad_start, axis=0, keepdims=True) + rank
    b_row = lax.broadcasted_iota(jnp.int32, (E, NBLK), 1)
    becount = jnp.sum((b_row >= blk_end).astype(jnp.int32), axis=0, keepdims=True)
    be_ref[...] = jnp.minimum(becount, E - 1)
    tot = jnp.sum(blk_counts)
    tot_ref[...] = jnp.broadcast_to(tot, (1, 1))
    valid = b_row[0:1] < tot
    xy_ref[...] = jnp.where(valid, b_row[0:1], NBLK - 1)


def _prep(gate):
    return pl.pallas_call(
        _prep_body,
        out_shape=(jax.ShapeDtypeStruct((1, T), jnp.int32),
                   jax.ShapeDtypeStruct((1, NBLK), jnp.int32),
                   jax.ShapeDtypeStruct((1, 1), jnp.int32),
                   jax.ShapeDtypeStruct((1, NBLK), jnp.int32)),
    )(gate.reshape(1, T))


@functools.lru_cache(maxsize=1)
def _sc_kernels():
    mesh = plsc.VectorSubcoreMesh(core_axis_name="c", subcore_axis_name="s")

    @functools.partial(
        pl.kernel,
        out_type=jax.ShapeDtypeStruct((TPAD, D_IN), jnp.float32),
        mesh=mesh,
        scratch_types=[
            pltpu.VMEM((CHUNK,), jnp.int32),
            pltpu.VMEM((CHUNK,), jnp.int32),
            pltpu.VMEM((CHUNK, D_IN), jnp.float32),
            pltpu.VMEM((CHUNK, D_IN), jnp.float32),
            pltpu.SemaphoreType.DMA,
            pltpu.SemaphoreType.DMA,
            pltpu.SemaphoreType.DMA,
            pltpu.SemaphoreType.DMA,
            pltpu.SemaphoreType.DMA,
            pltpu.SemaphoreType.DMA,
        ],
    )
    def scatter_rows(inp_hbm, dst_hbm, xpad_hbm,
                     idx0, idx1, rows0, rows1,
                     semi0, semi1, semr0, semr1, semw0, semw1):
        wid = lax.axis_index("s") * 2 + lax.axis_index("c")
        base = wid * TOK_PER_W
        idx, rows = (idx0, idx1), (rows0, rows1)
        semi, semr, semw = (semi0, semi1), (semr0, semr1), (semw0, semw1)
        for c in range(2):
            pltpu.async_copy(dst_hbm.at[pl.ds(base + c * CHUNK, CHUNK)], idx[c], semi[c])
            pltpu.async_copy(inp_hbm.at[pl.ds(base + c * CHUNK, CHUNK)], rows[c], semr[c])
        for c in range(NCHUNK):
            s = c % 2
            pltpu.make_async_copy(dst_hbm.at[pl.ds(base + c * CHUNK, CHUNK)], idx[s], semi[s]).wait()
            pltpu.make_async_copy(inp_hbm.at[pl.ds(base + c * CHUNK, CHUNK)], rows[s], semr[s]).wait()
            pltpu.async_copy(rows[s], xpad_hbm.at[idx[s]], semw[s])
            if c + 2 < NCHUNK:
                pltpu.make_async_copy(rows[s], xpad_hbm.at[idx[s]], semw[s]).wait()
                pltpu.async_copy(dst_hbm.at[pl.ds(base + (c + 2) * CHUNK, CHUNK)], idx[s], semi[s])
                pltpu.async_copy(inp_hbm.at[pl.ds(base + (c + 2) * CHUNK, CHUNK)], rows[s], semr[s])
        for c in range(NCHUNK - 2, NCHUNK):
            s = c % 2
            pltpu.make_async_copy(rows[s], xpad_hbm.at[idx[s]], semw[s]).wait()

    @functools.partial(
        pl.kernel,
        out_type=jax.ShapeDtypeStruct((T, D_OUT), jnp.float32),
        mesh=mesh,
        scratch_types=[
            pltpu.VMEM((TOK_PER_W,), jnp.int32),
            pltpu.VMEM((CHUNK, D_OUT), jnp.float32),
            pltpu.VMEM((CHUNK, D_OUT), jnp.float32),
            pltpu.SemaphoreType.DMA,
            pltpu.SemaphoreType.DMA,
            pltpu.SemaphoreType.DMA,
            pltpu.SemaphoreType.DMA,
        ],
    )
    def gather_rows(ypad_hbm, dst_hbm, out_hbm,
                    idxall, rows0, rows1,
                    semr0, semr1, semw0, semw1):
        wid = lax.axis_index("s") * 2 + lax.axis_index("c")
        base = wid * TOK_PER_W
        rows = (rows0, rows1)
        semr, semw = (semr0, semr1), (semw0, semw1)
        pltpu.sync_copy(dst_hbm.at[pl.ds(base, TOK_PER_W)], idxall)
        for c in range(2):
            pltpu.async_copy(ypad_hbm.at[idxall.at[pl.ds(c * CHUNK, CHUNK)]], rows[c], semr[c])
        for c in range(NCHUNK):
            s = c % 2
            pltpu.make_async_copy(ypad_hbm.at[idxall.at[pl.ds(c * CHUNK, CHUNK)]], rows[s], semr[s]).wait()
            pltpu.async_copy(rows[s], out_hbm.at[pl.ds(base + c * CHUNK, CHUNK)], semw[s])
            if c + 2 < NCHUNK:
                pltpu.make_async_copy(rows[s], out_hbm.at[pl.ds(base + c * CHUNK, CHUNK)], semw[s]).wait()
                pltpu.async_copy(ypad_hbm.at[idxall.at[pl.ds((c + 2) * CHUNK, CHUNK)]], rows[s], semr[s])
        for c in range(NCHUNK - 2, NCHUNK):
            s = c % 2
            pltpu.make_async_copy(rows[s], out_hbm.at[pl.ds(base + c * CHUNK, CHUNK)], semw[s]).wait()

    return scatter_rows, gather_rows


NH = 1


def _mm_body(be_ref, tot_ref, xy_ref, x_ref, w_ref, y_ref):
    b = pl.program_id(0)

    @pl.when(b < tot_ref[0])
    def _():
        y_ref[...] = lax.dot_general(
            x_ref[...], w_ref[0],
            dimension_numbers=(((1,), (1,)), ((), ())),
            preferred_element_type=jnp.float32,
        )


def _grouped_matmul(be, tot, xy, x_pad, weight):
    grid_spec = pltpu.PrefetchScalarGridSpec(
        num_scalar_prefetch=3,
        grid=(NBLK, NH),
        in_specs=[
            pl.BlockSpec((BLK, D_IN), lambda b, h, be_ref, tot_ref, xy_ref: (xy_ref[b], 0)),
            pl.BlockSpec((1, D_OUT // NH, D_IN), lambda b, h, be_ref, tot_ref, xy_ref: (be_ref[b], h, 0)),
        ],
        out_specs=pl.BlockSpec((BLK, D_OUT // NH), lambda b, h, be_ref, tot_ref, xy_ref: (xy_ref[b], h)),
    )
    return pl.pallas_call(
        _mm_body,
        grid_spec=grid_spec,
        out_shape=jax.ShapeDtypeStruct((TPAD, D_OUT), jnp.float32),
    )(be, tot, xy, x_pad, weight)


def kernel(inp, gate, weight):
    dst2, be2, tot2, xy2 = _prep(gate.astype(jnp.int32))
    dst = dst2.reshape(T)
    be = be2.reshape(NBLK)
    tot = tot2.reshape(1)
    xy = xy2.reshape(NBLK)
    scatter_rows, gather_rows = _sc_kernels()
    x_pad = scatter_rows(inp, dst)
    y_pad = _grouped_matmul(be, tot, xy, x_pad, weight)
    return gather_rows(y_pad, dst)


# speedup vs baseline: 1.0023x; 1.0023x over previous
import functools

import jax
import jax.numpy as jnp
from jax import lax
from jax.experimental import pallas as pl
from jax.experimental.pallas import tpu as pltpu
from jax.experimental.pallas import tpu_sc as plsc

E = 64
D_IN = 768
D_OUT = 768
T = 8192
BLK = 256
NBLK = T // BLK + E
TPAD = NBLK * BLK

NW = 32
TOK_PER_W = T // NW
CHUNK = 128
NCHUNK = TOK_PER_W // CHUNK


def _cumsum_lanes(x, n):
    i = lax.broadcasted_iota(jnp.int32, x.shape, len(x.shape) - 1)
    k = 1
    while k < n:
        r = pltpu.roll(x, k, axis=len(x.shape) - 1)
        x = x + jnp.where(i >= k, r, 0)
        k *= 2
    return x


def _cumsum_sublanes(x, n):
    i = lax.broadcasted_iota(jnp.int32, x.shape, 0)
    k = 1
    while k < n:
        r = pltpu.roll(x, k, axis=0)
        x = x + jnp.where(i >= k, r, 0)
        k *= 2
    return x


def _prep_body(gate_ref, dst_ref, be_ref, tot_ref, xy_ref):
    gate = gate_ref[...]
    e_col = lax.broadcasted_iota(jnp.int32, (E, T), 0)
    onehot = (gate == e_col).astype(jnp.int32)
    cum = _cumsum_lanes(onehot, T)
    rank = jnp.sum(onehot * cum, axis=0, keepdims=True) - 1
    counts = cum[:, T - 1:T]
    blk_counts = (counts + BLK - 1) // BLK
    blk_end = _cumsum_sublanes(blk_counts, E)
    blk_start = blk_end - blk_counts
    pad_start = blk_start * BLK
    dst_ref[...] = jnp.sum(onehot * pad_start, axis=0, keepdims=True) + rank
    b_row = lax.broadcasted_iota(jnp.int32, (E, NBLK), 1)
    becount = jnp.sum((b_row >= blk_end).astype(jnp.int32), axis=0, keepdims=True)
    be_ref[...] = jnp.minimum(becount, E - 1)
    tot = jnp.sum(blk_counts)
    tot_ref[...] = jnp.broadcast_to(tot, (1, 1))
    valid = b_row[0:1] < tot
    xy_ref[...] = jnp.where(valid, b_row[0:1], NBLK - 1)


def _prep(gate):
    return pl.pallas_call(
        _prep_body,
        out_shape=(jax.ShapeDtypeStruct((1, T), jnp.int32),
                   jax.ShapeDtypeStruct((1, NBLK), jnp.int32),
                   jax.ShapeDtypeStruct((1, 1), jnp.int32),
                   jax.ShapeDtypeStruct((1, NBLK), jnp.int32)),
    )(gate.reshape(1, T))


@functools.lru_cache(maxsize=1)
def _sc_kernels():
    mesh = plsc.VectorSubcoreMesh(core_axis_name="c", subcore_axis_name="s")

    @functools.partial(
        pl.kernel,
        out_type=jax.ShapeDtypeStruct((TPAD, D_IN), jnp.float32),
        mesh=mesh,
        scratch_types=[
            pltpu.VMEM((CHUNK,), jnp.int32),
            pltpu.VMEM((CHUNK, D_IN), jnp.float32),
            pltpu.SemaphoreType.DMA,
        ],
    )
    def scatter_rows(inp_hbm, dst_hbm, xpad_hbm, idx_v, rows_v, sem):
        wid = lax.axis_index("s") * 2 + lax.axis_index("c")
        for c in range(NCHUNK):
            base = wid * TOK_PER_W + c * CHUNK
            pltpu.sync_copy(dst_hbm.at[pl.ds(base, CHUNK)], idx_v)
            pltpu.sync_copy(inp_hbm.at[pl.ds(base, CHUNK)], rows_v)
            pltpu.async_copy(rows_v, xpad_hbm.at[idx_v], sem).wait()

    @functools.partial(
        pl.kernel,
        out_type=jax.ShapeDtypeStruct((T, D_OUT), jnp.float32),
        mesh=mesh,
        scratch_types=[
            pltpu.VMEM((CHUNK,), jnp.int32),
            pltpu.VMEM((CHUNK, D_OUT), jnp.float32),
            pltpu.SemaphoreType.DMA,
        ],
    )
    def gather_rows(ypad_hbm, dst_hbm, out_hbm, idx_v, rows_v, sem):
        wid = lax.axis_index("s") * 2 + lax.axis_index("c")
        for c in range(NCHUNK):
            base = wid * TOK_PER_W + c * CHUNK
            pltpu.sync_copy(dst_hbm.at[pl.ds(base, CHUNK)], idx_v)
            pltpu.async_copy(ypad_hbm.at[idx_v], rows_v, sem).wait()
            pltpu.sync_copy(rows_v, out_hbm.at[pl.ds(base, CHUNK)])

    return scatter_rows, gather_rows


NH = 1


def _mm_body(be_ref, tot_ref, xy_ref, x_ref, w_ref, y_ref):
    b = pl.program_id(0)

    @pl.when(b < tot_ref[0])
    def _():
        y_ref[...] = lax.dot_general(
            x_ref[...], w_ref[0],
            dimension_numbers=(((1,), (1,)), ((), ())),
            preferred_element_type=jnp.float32,
        )


def _grouped_matmul(be, tot, xy, x_pad, weight):
    grid_spec = pltpu.PrefetchScalarGridSpec(
        num_scalar_prefetch=3,
        grid=(NBLK, NH),
        in_specs=[
            pl.BlockSpec((BLK, D_IN), lambda b, h, be_ref, tot_ref, xy_ref: (xy_ref[b], 0)),
            pl.BlockSpec((1, D_OUT // NH, D_IN), lambda b, h, be_ref, tot_ref, xy_ref: (be_ref[b], h, 0)),
        ],
        out_specs=pl.BlockSpec((BLK, D_OUT // NH), lambda b, h, be_ref, tot_ref, xy_ref: (xy_ref[b], h)),
    )
    return pl.pallas_call(
        _mm_body,
        grid_spec=grid_spec,
        out_shape=jax.ShapeDtypeStruct((TPAD, D_OUT), jnp.float32),
    )(be, tot, xy, x_pad, weight)


def kernel(inp, gate, weight):
    dst2, be2, tot2, xy2 = _prep(gate.astype(jnp.int32))
    dst = dst2.reshape(T)
    be = be2.reshape(NBLK)
    tot = tot2.reshape(1)
    xy = xy2.reshape(NBLK)
    scatter_rows, gather_rows = _sc_kernels()
    x_pad = scatter_rows(inp, dst)
    y_pad = _grouped_matmul(be, tot, xy, x_pad, weight)
    return gather_rows(y_pad, dst)
